# baseline (speedup 1.0000x reference)
import functools
import math

import jax
import jax.numpy as jnp
from jax import lax
from jax.experimental import pallas as pl
from jax.experimental.pallas import tpu as pltpu
from jax.experimental.pallas import tpu_sc as plsc

D = 128
CHUNK = 256
SCALE = math.sqrt(float(D))
UNROLL = 8


@functools.lru_cache(maxsize=None)
def _embed_kernel(n_rows):
    info = plsc.get_sparse_core_info()
    nw = info.num_cores * info.num_subcores
    per_w = n_rows // nw
    n_chunks = per_w // CHUNK
    assert per_w * nw == n_rows and n_chunks * CHUNK == per_w
    n_super = (n_chunks - 4) // 3 + 1
    assert 3 * (n_super - 1) + 4 == n_chunks
    mesh = plsc.VectorSubcoreMesh(core_axis_name="c", subcore_axis_name="s")

    @functools.partial(
        pl.kernel,
        mesh=mesh,
        out_type=jax.ShapeDtypeStruct((n_rows, D), jnp.float32),
        scratch_types=[
            pltpu.VMEM((per_w,), jnp.int32),
            pltpu.VMEM((3, CHUNK, D), jnp.float32),
            pltpu.SemaphoreType.DMA,
            pltpu.SemaphoreType.DMA,
            pltpu.SemaphoreType.DMA,
            pltpu.SemaphoreType.DMA,
            pltpu.SemaphoreType.DMA,
            pltpu.SemaphoreType.DMA,
            pltpu.SemaphoreType.DMA,
        ],
    )
    def k(table_hbm, idx_hbm, out_hbm, idx_v, rows_v, isem,
          g0, g1, g2, s0, s1, s2):
        wid = lax.axis_index("s") * info.num_cores + lax.axis_index("c")
        base = wid * per_w
        pltpu.async_copy(idx_hbm.at[pl.ds(base, per_w)], idx_v, isem).wait()
        gsem = (g0, g1, g2)
        ssem = (s0, s1, s2)

        def fire_gather(c, slot):
            pltpu.async_copy(
                table_hbm.at[idx_v.at[pl.ds(c * CHUNK, CHUNK)]],
                rows_v.at[slot], gsem[slot])

        def drain_gather(slot):
            pltpu.make_async_copy(
                table_hbm.at[idx_v.at[pl.ds(0, CHUNK)]],
                rows_v.at[slot], gsem[slot]).wait()

        def fire_store(c, slot):
            pltpu.async_copy(
                rows_v.at[slot],
                out_hbm.at[pl.ds(base + c * CHUNK, CHUNK)], ssem[slot])

        def drain_store(slot):
            pltpu.make_async_copy(
                rows_v.at[slot],
                out_hbm.at[pl.ds(base, CHUNK)], ssem[slot]).wait()

        def scale(slot):
            def sbody(it, carry):
                j = it * UNROLL
                for u in range(UNROLL):
                    for i in range(D // 16):
                        sl = pl.ds(i * 16, 16)
                        rows_v[slot, j + u, sl] = (
                            rows_v[slot, j + u, sl] * SCALE)
                return carry
            lax.fori_loop(0, CHUNK // UNROLL, sbody, 0)

        def retire(c_prev, slot_prev):
            drain_gather(slot_prev)
            scale(slot_prev)
            fire_store(c_prev, slot_prev)

        fire_gather(0, 0)
        fire_gather(1, 1)
        retire(0, 0)
        fire_gather(2, 2)
        retire(1, 1)

        def body(gg, carry):
            for r in range(3):
                c = gg * 3 + r
                drain_store(r)
                fire_gather(c, r)
                retire(c - 1, (r + 2) % 3)
            return carry

        lax.fori_loop(1, n_super, body, 0)

        c_last = n_chunks - 1
        drain_store(c_last % 3)
        fire_gather(c_last, c_last % 3)
        retire(c_last - 1, (c_last + 2) % 3)
        retire(c_last, c_last % 3)
        for slot in range(3):
            drain_store(slot)

    return k


def kernel(x, table):
    b, s = x.shape
    xf = x.reshape(b * s)
    out = _embed_kernel(b * s)(table, xf)
    return out.reshape(b, s, D)
